# Initial kernel scaffold; baseline (speedup 1.0000x reference)
import jax
import jax.numpy as jnp
from jax.experimental import pallas as pl


def kernel(inputs):
    raise NotImplementedError("write your pallas kernel here")


# speedup vs baseline: 44.7249x; 44.7249x over previous
import jax
import jax.numpy as jnp
from jax.experimental import pallas as pl


def _bucket_body(x_ref, o_ref):
    x = x_ref[...]
    f = jnp.floor(x * 4.0)
    f = jnp.clip(f, -17.0, 16.0)
    o_ref[...] = f.astype(jnp.int32) + 17


def kernel(inputs):
    n, m = inputs.shape
    rows = 512
    out = pl.pallas_call(
        _bucket_body,
        grid=(n // rows,),
        in_specs=[pl.BlockSpec((rows, m), lambda i: (i, 0))],
        out_specs=pl.BlockSpec((rows, m), lambda i: (i, 0)),
        out_shape=jax.ShapeDtypeStruct((n, m), jnp.int32),
    )(inputs)
    return out.astype(jnp.int64)
